# Initial kernel scaffold; baseline (speedup 1.0000x reference)
import functools

import jax
import jax.numpy as jnp
from jax import lax
from jax.experimental import pallas as pl
from jax.experimental.pallas import tpu as pltpu
from jax.experimental.pallas import tpu_sc as plsc

N = 10000
E = 320000
D = 128
H1, C1 = 3, 64
H2, C2 = 1, 128
HC1 = H1 * C1
HC2 = H2 * C2
ROWW1 = 208
ROWW2 = 160

NC, NS = 2, 16
NT = NC * NS
EPT = E // NT
K = 80
J = K // 16
NCHUNK = EPT // K
RPT = N // NS

_F32 = jnp.float32


def _edge_pass_body(H, C, ROWW, extras,
                    src_hbm, dst_hbm, ea_hbm, xl_hbm, xr_hbm,
                    attsp_hbm, wesp_hbm, z_hbm, out_hbm,
                    src_v, dst_v, ea_v, xl_rows, xr_rows, contrib,
                    attsp_v, wesp_v, accum, sem1, sem2):
    HC = H * C
    core = lax.axis_index("c")
    sub = lax.axis_index("s")
    wid = core * NS + sub

    pltpu.sync_copy(attsp_hbm, attsp_v)
    pltpu.sync_copy(wesp_hbm, wesp_v)
    pltpu.sync_copy(z_hbm.at[pl.ds(sub * RPT, RPT)],
                    accum.at[pl.ds(sub * RPT, RPT)])

    zero16 = jnp.zeros((16,), _F32)
    ones16 = jnp.ones((16,), _F32)
    for r in range(K):
        for tc0 in range(HC, ROWW, 16):
            contrib[r, pl.ds(tc0, 16)] = zero16

    iota16 = lax.iota(jnp.int32, 16)
    eids = [iota16 + (j * 16) for j in range(J)]

    plsc.subcore_barrier()

    ebase = wid * EPT

    def chunk(g, carry):
        base = ebase + g * K
        pltpu.sync_copy(src_hbm.at[pl.ds(base, K)], src_v)
        pltpu.sync_copy(dst_hbm.at[pl.ds(base, K)], dst_v)
        pltpu.sync_copy(ea_hbm.at[pl.ds(base, K)], ea_v)
        cp1 = pltpu.async_copy(xl_hbm.at[src_v], xl_rows, sem1)
        cp2 = pltpu.async_copy(xr_hbm.at[dst_v], xr_rows, sem2)
        cp1.wait()
        cp2.wait()

        eav = [ea_v[pl.ds(j * 16, 16)] for j in range(J)]

        w = []
        for h in range(H):
            def cbody(ci, lg, _h=h):
                c = _h * C + ci
                cvec = jnp.full((16,), c, jnp.int32)
                attc = plsc.load_gather(attsp_v, [cvec, iota16])
                wec = plsc.load_gather(wesp_v, [cvec, iota16])
                out = []
                for j in range(J):
                    xlv = plsc.load_gather(xl_rows, [eids[j], cvec])
                    xrv = plsc.load_gather(xr_rows, [eids[j], cvec])
                    s = xlv + xrv + eav[j] * wec
                    lk = jnp.maximum(s, 0.2 * s)
                    out.append(lg[j] + attc * lk)
                return tuple(out)
            lg = lax.fori_loop(0, C, cbody, tuple(zero16 for _ in range(J)))
            w.append([jnp.exp(lg[j]) for j in range(J)])

        for h in range(H):
            def c2body(ci, carry2, _h=h):
                c = _h * C + ci
                cvec = jnp.full((16,), c, jnp.int32)
                for j in range(J):
                    xlv = plsc.load_gather(xl_rows, [eids[j], cvec])
                    plsc.store_scatter(contrib, [eids[j], cvec],
                                       xlv * w[_h][j])
                return carry2
            lax.fori_loop(0, C, c2body, 0)

        for h in range(H):
            hcol = jnp.full((16,), HC + h, jnp.int32)
            for j in range(J):
                plsc.store_scatter(contrib, [eids[j], hcol], w[h][j])
        if extras:
            eacol = jnp.full((16,), HC + H, jnp.int32)
            degcol = jnp.full((16,), HC + H + 1, jnp.int32)
            for j in range(J):
                plsc.store_scatter(contrib, [eids[j], eacol], eav[j])
                plsc.store_scatter(contrib, [eids[j], degcol], ones16)

        pltpu.sync_copy(contrib, accum.at[dst_v], add=True)
        return carry

    lax.fori_loop(0, NCHUNK, chunk, 0)

    plsc.subcore_barrier()
    pltpu.sync_copy(accum.at[pl.ds(sub * RPT, RPT)],
                    out_hbm.at[core, pl.ds(sub * RPT, RPT)])


def _make_edge_pass(H, C, ROWW, extras):
    HC = H * C
    mesh = plsc.VectorSubcoreMesh(core_axis_name="c", subcore_axis_name="s",
                                  num_cores=NC, num_subcores=NS)
    return pl.kernel(
        functools.partial(_edge_pass_body, H, C, ROWW, extras),
        out_type=jax.ShapeDtypeStruct((NC, N, ROWW), _F32),
        mesh=mesh,
        scratch_types=[
            pltpu.VMEM((K,), jnp.int32),
            pltpu.VMEM((K,), jnp.int32),
            pltpu.VMEM((K,), _F32),
            pltpu.VMEM((K, HC), _F32),
            pltpu.VMEM((K, HC), _F32),
            pltpu.VMEM((K, ROWW), _F32),
            pltpu.VMEM((HC, 16), _F32),
            pltpu.VMEM((HC, 16), _F32),
            pltpu.VMEM_SHARED((N, ROWW), _F32),
            pltpu.SemaphoreType.DMA,
            pltpu.SemaphoreType.DMA,
        ],
    )


def _mm_pair(x, Wl, bl, Wr, br):
    M, Kd = x.shape
    HC = Wl.shape[1]
    BM = 1000

    def body(x_ref, wl_ref, bl_ref, wr_ref, br_ref, ol_ref, or_ref):
        xb = x_ref[...]
        ol_ref[...] = jnp.dot(xb, wl_ref[...],
                              preferred_element_type=_F32) + bl_ref[...]
        or_ref[...] = jnp.dot(xb, wr_ref[...],
                              preferred_element_type=_F32) + br_ref[...]

    return pl.pallas_call(
        body,
        grid=(M // BM,),
        in_specs=[
            pl.BlockSpec((BM, Kd), lambda i: (i, 0)),
            pl.BlockSpec((Kd, HC), lambda i: (0, 0)),
            pl.BlockSpec((1, HC), lambda i: (0, 0)),
            pl.BlockSpec((Kd, HC), lambda i: (0, 0)),
            pl.BlockSpec((1, HC), lambda i: (0, 0)),
        ],
        out_specs=[
            pl.BlockSpec((BM, HC), lambda i: (i, 0)),
            pl.BlockSpec((BM, HC), lambda i: (i, 0)),
        ],
        out_shape=[
            jax.ShapeDtypeStruct((M, HC), _F32),
            jax.ShapeDtypeStruct((M, HC), _F32),
        ],
    )(x, Wl, bl, Wr, br)


def _finalize(acc0, acc1, xl, xr, la_in, Wef, attf, biasf, sel2, seldn,
              H, C, ROWW, layer1):
    HC = H * C
    BR = 1000

    def body(*refs):
        if layer1:
            (a0_ref, a1_ref, xl_ref, xr_ref, wef_ref, attf_ref, b_ref,
             s2_ref, sd_ref, out_ref, la_ref) = refs
        else:
            (a0_ref, a1_ref, xl_ref, xr_ref, lain_ref, wef_ref, attf_ref,
             b_ref, s2_ref, sd_ref, out_ref) = refs
        acc = a0_ref[...] + a1_ref[...]
        S = acc[:, :HC]
        Dh = acc[:, HC:HC + H]
        if layer1:
            la = acc[:, HC + H:HC + H + 1] / jnp.maximum(
                acc[:, HC + H + 1:HC + H + 2], 1.0)
        else:
            la = lain_ref[...]
        xlb = xl_ref[...]
        m = xlb + xr_ref[...] + la * wef_ref[...]
        m = jnp.where(m > 0, m, 0.2 * m)
        a = jnp.dot(m * attf_ref[...], s2_ref[...],
                    preferred_element_type=_F32)
        wl = jnp.exp(a)
        dfull = jnp.dot(Dh + wl, sd_ref[...], preferred_element_type=_F32)
        wfull = jnp.dot(wl, sd_ref[...], preferred_element_type=_F32)
        o = (S + xlb * wfull) / (dfull + 1e-16) + b_ref[...]
        out_ref[...] = jnp.where(o > 0, o, jnp.exp(o) - 1.0)
        if layer1:
            la_ref[...] = la

    const = lambda i: (0, 0)
    row = lambda i: (i, 0)
    in_specs = [
        pl.BlockSpec((BR, ROWW), row),
        pl.BlockSpec((BR, ROWW), row),
        pl.BlockSpec((BR, HC), row),
        pl.BlockSpec((BR, HC), row),
    ]
    args = [acc0, acc1, xl, xr]
    if not layer1:
        in_specs.append(pl.BlockSpec((BR, 1), row))
        args.append(la_in)
    in_specs += [
        pl.BlockSpec((1, HC), const),
        pl.BlockSpec((1, HC), const),
        pl.BlockSpec((1, HC), const),
        pl.BlockSpec((HC, H), const),
        pl.BlockSpec((H, HC), const),
    ]
    args += [Wef, attf, biasf, sel2, seldn]
    out_specs = [pl.BlockSpec((BR, HC), row)]
    out_shape = [jax.ShapeDtypeStruct((N, HC), _F32)]
    if layer1:
        out_specs.append(pl.BlockSpec((BR, 1), row))
        out_shape.append(jax.ShapeDtypeStruct((N, 1), _F32))
    res = pl.pallas_call(body, grid=(N // BR,), in_specs=in_specs,
                         out_specs=out_specs, out_shape=out_shape)(*args)
    return res if layer1 else res[0]


_edge_pass1 = _make_edge_pass(H1, C1, ROWW1, True)
_edge_pass2 = _make_edge_pass(H2, C2, ROWW2, False)


def _head_mats(H, C):
    eye = jnp.eye(H, dtype=_F32)
    sel2 = jnp.repeat(eye, C, axis=0)
    seldn = jnp.repeat(eye, C, axis=1)
    return sel2, seldn


def kernel(x, edge_index, edge_attr, Wl1, bl1, Wr1, br1, We1, att1, bias1,
           Wl2, bl2, Wr2, br2, We2, att2, bias2):
    src = edge_index[0]
    dst = edge_index[1]
    eaf = edge_attr[:, 0]

    attf1 = att1.reshape(1, HC1)
    wef1 = We1.reshape(1, HC1)
    attf2 = att2.reshape(1, HC2)
    wef2 = We2.reshape(1, HC2)
    attsp1 = jnp.broadcast_to(att1.reshape(HC1, 1), (HC1, 16))
    wesp1 = jnp.broadcast_to(We1.reshape(HC1, 1), (HC1, 16))
    attsp2 = jnp.broadcast_to(att2.reshape(HC2, 1), (HC2, 16))
    wesp2 = jnp.broadcast_to(We2.reshape(HC2, 1), (HC2, 16))
    z1 = jnp.zeros((N, ROWW1), _F32)
    z2 = jnp.zeros((N, ROWW2), _F32)
    sel2_1, seldn_1 = _head_mats(H1, C1)
    sel2_2, seldn_2 = _head_mats(H2, C2)

    xl1, xr1 = _mm_pair(x, Wl1, bl1.reshape(1, HC1), Wr1, br1.reshape(1, HC1))
    acc1 = _edge_pass1(src, dst, eaf, xl1, xr1, attsp1, wesp1, z1)
    h1, la = _finalize(acc1[0], acc1[1], xl1, xr1, None,
                       wef1, attf1, bias1.reshape(1, HC1),
                       sel2_1, seldn_1, H1, C1, ROWW1, True)
    xl2, xr2 = _mm_pair(h1, Wl2, bl2.reshape(1, HC2), Wr2, br2.reshape(1, HC2))
    acc2 = _edge_pass2(src, dst, eaf, xl2, xr2, attsp2, wesp2, z2)
    h2 = _finalize(acc2[0], acc2[1], xl2, xr2, la,
                   wef2, attf2, bias2.reshape(1, HC2),
                   sel2_2, seldn_2, H2, C2, ROWW2, False)
    return h2


# speedup vs baseline: 4.0220x; 4.0220x over previous
import functools

import jax
import jax.numpy as jnp
from jax import lax
from jax.experimental import pallas as pl
from jax.experimental.pallas import tpu as pltpu
from jax.experimental.pallas import tpu_sc as plsc

N = 10000
E = 320000
D = 128
H1, C1 = 3, 64
H2, C2 = 1, 128
HC1 = H1 * C1
HC2 = H2 * C2

NC, NS = 2, 16
NT = NC * NS
K = 80
J = K // 16
RPT8 = 624
REM = N - NS * RPT8
SW = 8

_F32 = jnp.float32
_SC_PARAMS = dict(use_tc_tiling_on_sc=False, needs_layout_passes=False)


def _zero_accum(stage, accum, sub):
    base = sub * RPT8
    for b in range(7):
        pltpu.sync_copy(stage, accum.at[pl.ds(base + b * 80, 80)])
    pltpu.sync_copy(stage.at[pl.ds(0, 64)], accum.at[pl.ds(base + 560, 64)])

    @pl.when(sub == 0)
    def _rem():
        pltpu.sync_copy(stage.at[pl.ds(0, REM)],
                        accum.at[pl.ds(NS * RPT8, REM)])


def _flush_accum(accum, out_hbm, stage, core, sub):
    base = sub * RPT8
    for b in range(7):
        pltpu.sync_copy(accum.at[pl.ds(base + b * 80, 80)], stage)
        pltpu.sync_copy(stage, out_hbm.at[core, pl.ds(base + b * 80, 80)])
    pltpu.sync_copy(accum.at[pl.ds(base + 560, 64)], stage.at[pl.ds(0, 64)])
    pltpu.sync_copy(stage.at[pl.ds(0, 64)],
                    out_hbm.at[core, pl.ds(base + 560, 64)])

    @pl.when(sub == 0)
    def _rem():
        pltpu.sync_copy(accum.at[pl.ds(NS * RPT8, REM)],
                        stage.at[pl.ds(0, REM)])
        pltpu.sync_copy(stage.at[pl.ds(0, REM)],
                        out_hbm.at[core, pl.ds(NS * RPT8, REM)])


def _pass_a_body(H, C,
                 src_hbm, dst_hbm, ea_hbm, xl_hbm, xr_hbm,
                 attsp_hbm, wesp_hbm, acc_hbm, w_hbm,
                 src_v, dst_v, ea_v, xl_rows, xr_rows, wsb,
                 attsp_v, wesp_v, accum, sem1, sem2):
    HC = H * C
    core = lax.axis_index("c")
    sub = lax.axis_index("s")
    wid = core * NS + sub
    ept = E // NT

    pltpu.sync_copy(attsp_hbm, attsp_v)
    pltpu.sync_copy(wesp_hbm, wesp_v)

    zero16 = jnp.zeros((16,), _F32)
    ones16 = jnp.ones((16,), _F32)
    iota16 = lax.iota(jnp.int32, 16)
    eids = [iota16 + (j * 16) for j in range(J)]

    for j in range(J):
        for col in range(SW):
            plsc.store_scatter(wsb, [eids[j], jnp.full((16,), col, jnp.int32)],
                               zero16)
    _zero_accum(wsb, accum, sub)

    plsc.subcore_barrier()

    ebase = wid * ept

    def chunk(g, carry):
        base = ebase + g * K
        pltpu.sync_copy(src_hbm.at[pl.ds(base, K)], src_v)
        pltpu.sync_copy(dst_hbm.at[pl.ds(base, K)], dst_v)
        pltpu.sync_copy(ea_hbm.at[pl.ds(base, K)], ea_v)
        cp1 = pltpu.async_copy(xl_hbm.at[src_v], xl_rows, sem1)
        cp2 = pltpu.async_copy(xr_hbm.at[dst_v], xr_rows, sem2)
        cp1.wait()
        cp2.wait()

        eav = [ea_v[pl.ds(j * 16, 16)] for j in range(J)]

        for h in range(H):
            def cbody(ci, lg, _h=h):
                c = _h * C + ci
                cvec = jnp.full((16,), c, jnp.int32)
                attc = attsp_v[c]
                wec = wesp_v[c]
                out = []
                for j in range(J):
                    xlv = plsc.load_gather(xl_rows, [eids[j], cvec])
                    xrv = plsc.load_gather(xr_rows, [eids[j], cvec])
                    s = xlv + xrv + eav[j] * wec
                    lk = jnp.maximum(s, 0.2 * s)
                    out.append(lg[j] + attc * lk)
                return tuple(out)
            lg = lax.fori_loop(0, C, cbody, tuple(zero16 for _ in range(J)))
            hcol = jnp.full((16,), h, jnp.int32)
            for j in range(J):
                plsc.store_scatter(wsb, [eids[j], hcol], jnp.exp(lg[j]))

        eacol = jnp.full((16,), H, jnp.int32)
        onecol = jnp.full((16,), H + 1, jnp.int32)
        for j in range(J):
            plsc.store_scatter(wsb, [eids[j], eacol], eav[j])
            plsc.store_scatter(wsb, [eids[j], onecol], ones16)

        pltpu.sync_copy(wsb, w_hbm.at[pl.ds(base, K)])
        pltpu.sync_copy(wsb, accum.at[dst_v], add=True)
        return carry

    lax.fori_loop(0, ept // K, chunk, 0)

    plsc.subcore_barrier()
    _flush_accum(accum, acc_hbm, wsb, core, sub)


@functools.lru_cache(maxsize=None)
def _make_pass_a(H, C):
    HC = H * C
    mesh = plsc.VectorSubcoreMesh(core_axis_name="c", subcore_axis_name="s",
                                  num_cores=NC, num_subcores=NS)
    return pl.kernel(
        functools.partial(_pass_a_body, H, C),
        out_type=(jax.ShapeDtypeStruct((NC, N, SW), _F32),
                  jax.ShapeDtypeStruct((E, SW), _F32)),
        mesh=mesh,
        compiler_params=pltpu.CompilerParams(**_SC_PARAMS),
        scratch_types=[
            pltpu.VMEM((K,), jnp.int32),
            pltpu.VMEM((K,), jnp.int32),
            pltpu.VMEM((K,), _F32),
            pltpu.VMEM((K, HC), _F32),
            pltpu.VMEM((K, HC), _F32),
            pltpu.VMEM((K, SW), _F32),
            pltpu.VMEM((HC, 16), _F32),
            pltpu.VMEM((HC, 16), _F32),
            pltpu.VMEM_SHARED((N, SW), _F32),
            pltpu.SemaphoreType.DMA,
            pltpu.SemaphoreType.DMA,
        ],
    )


def _pass_b_body(WA, WB, wcol0, wcol1,
                 src_hbm, dst_hbm, ta0_hbm, ta1_hbm, tb0_hbm, tb1_hbm,
                 w_hbm, acc_hbm,
                 src_v, dst_v, bufa, bufb, wvb, contrib, accum, sem1, sem2):
    ACCW = WA + WB
    core = lax.axis_index("c")
    sub = lax.axis_index("s")
    ept = E // NS

    zero16 = jnp.zeros((16,), _F32)
    for r in range(K):
        for c0 in range(0, ACCW, 16):
            contrib[r, pl.ds(c0, 16)] = zero16
    _zero_accum(contrib, accum, sub)

    iota16 = lax.iota(jnp.int32, 16)
    eids = [iota16 + (j * 16) for j in range(J)]
    wca = jnp.where(core == 0, wcol0, wcol1)
    wcavec = jnp.full((16,), wca, jnp.int32)

    plsc.subcore_barrier()

    ebase = sub * ept

    def chunk(g, carry):
        base = ebase + g * K
        pltpu.sync_copy(src_hbm.at[pl.ds(base, K)], src_v)
        pltpu.sync_copy(dst_hbm.at[pl.ds(base, K)], dst_v)
        pltpu.sync_copy(w_hbm.at[pl.ds(base, K)], wvb)

        @pl.when(core == 0)
        def _g0():
            cp1 = pltpu.async_copy(ta0_hbm.at[src_v], bufa, sem1)
            if WB:
                cp2 = pltpu.async_copy(tb0_hbm.at[src_v], bufb, sem2)
                cp2.wait()
            cp1.wait()

        @pl.when(core != 0)
        def _g1():
            cp1 = pltpu.async_copy(ta1_hbm.at[src_v], bufa, sem1)
            if WB:
                cp2 = pltpu.async_copy(tb1_hbm.at[src_v], bufb, sem2)
                cp2.wait()
            cp1.wait()

        wa = [plsc.load_gather(wvb, [eids[j], wcavec]) for j in range(J)]

        def abody(ci, carry2):
            cvec = jnp.full((16,), ci, jnp.int32)
            for j in range(J):
                xlv = plsc.load_gather(bufa, [eids[j], cvec])
                plsc.store_scatter(contrib, [eids[j], cvec], xlv * wa[j])
            return carry2
        lax.fori_loop(0, WA, abody, 0)

        if WB:
            wbv = jnp.full((16,), wcol1 if wcol0 == wcol1 else 1, jnp.int32)
            wb = [plsc.load_gather(wvb, [eids[j], wbv]) for j in range(J)]

            def bbody(ci, carry2):
                cvec = jnp.full((16,), ci, jnp.int32)
                ovec = jnp.full((16,), WA + ci, jnp.int32)
                for j in range(J):
                    xlv = plsc.load_gather(bufb, [eids[j], cvec])
                    plsc.store_scatter(contrib, [eids[j], ovec], xlv * wb[j])
                return carry2
            lax.fori_loop(0, WB, bbody, 0)

        pltpu.sync_copy(contrib, accum.at[dst_v], add=True)
        return carry

    lax.fori_loop(0, ept // K, chunk, 0)

    plsc.subcore_barrier()
    _flush_accum(accum, acc_hbm, contrib, core, sub)


@functools.lru_cache(maxsize=None)
def _make_pass_b(WA, WB, wcol0, wcol1):
    ACCW = WA + WB
    mesh = plsc.VectorSubcoreMesh(core_axis_name="c", subcore_axis_name="s",
                                  num_cores=NC, num_subcores=NS)
    return pl.kernel(
        functools.partial(_pass_b_body, WA, WB, wcol0, wcol1),
        out_type=jax.ShapeDtypeStruct((NC, N, ACCW), _F32),
        mesh=mesh,
        compiler_params=pltpu.CompilerParams(**_SC_PARAMS),
        scratch_types=[
            pltpu.VMEM((K,), jnp.int32),
            pltpu.VMEM((K,), jnp.int32),
            pltpu.VMEM((K, WA), _F32),
            pltpu.VMEM((K, max(WB, 8)), _F32),
            pltpu.VMEM((K, SW), _F32),
            pltpu.VMEM((K, ACCW), _F32),
            pltpu.VMEM_SHARED((N, ACCW), _F32),
            pltpu.SemaphoreType.DMA,
            pltpu.SemaphoreType.DMA,
        ],
    )


def _mm_pair(x, Wl, bl, Wr, br):
    M, Kd = x.shape
    HC = Wl.shape[1]
    BM = 1000

    def body(x_ref, wl_ref, bl_ref, wr_ref, br_ref, ol_ref, or_ref):
        xb = x_ref[...]
        ol_ref[...] = jnp.dot(xb, wl_ref[...],
                              preferred_element_type=_F32) + bl_ref[...]
        or_ref[...] = jnp.dot(xb, wr_ref[...],
                              preferred_element_type=_F32) + br_ref[...]

    return pl.pallas_call(
        body,
        grid=(M // BM,),
        in_specs=[
            pl.BlockSpec((BM, Kd), lambda i: (i, 0)),
            pl.BlockSpec((Kd, HC), lambda i: (0, 0)),
            pl.BlockSpec((1, HC), lambda i: (0, 0)),
            pl.BlockSpec((Kd, HC), lambda i: (0, 0)),
            pl.BlockSpec((1, HC), lambda i: (0, 0)),
        ],
        out_specs=[
            pl.BlockSpec((BM, HC), lambda i: (i, 0)),
            pl.BlockSpec((BM, HC), lambda i: (i, 0)),
        ],
        out_shape=[
            jax.ShapeDtypeStruct((M, HC), _F32),
            jax.ShapeDtypeStruct((M, HC), _F32),
        ],
    )(x, Wl, bl, Wr, br)


def _finalize(a0, a1, S, xl, xr, la_in, Wef, attf, biasf, sel2, seldn,
              H, C, layer1):
    HC = H * C
    BR = 1000

    def body(*refs):
        if layer1:
            (a0_ref, a1_ref, s_ref, xl_ref, xr_ref, wef_ref, attf_ref,
             b_ref, s2_ref, sd_ref, out_ref, la_ref) = refs
        else:
            (a0_ref, a1_ref, s_ref, xl_ref, xr_ref, lain_ref, wef_ref,
             attf_ref, b_ref, s2_ref, sd_ref, out_ref) = refs
        stats = a0_ref[...] + a1_ref[...]
        S = s_ref[...]
        Dh = stats[:, :H]
        if layer1:
            la = stats[:, H:H + 1] / jnp.maximum(stats[:, H + 1:H + 2], 1.0)
        else:
            la = lain_ref[...]
        xlb = xl_ref[...]
        m = xlb + xr_ref[...] + la * wef_ref[...]
        m = jnp.where(m > 0, m, 0.2 * m)
        a = jnp.dot(m * attf_ref[...], s2_ref[...],
                    preferred_element_type=_F32)
        wl = jnp.exp(a)
        dfull = jnp.dot(Dh + wl, sd_ref[...], preferred_element_type=_F32)
        wfull = jnp.dot(wl, sd_ref[...], preferred_element_type=_F32)
        o = (S + xlb * wfull) / (dfull + 1e-16) + b_ref[...]
        out_ref[...] = jnp.where(o > 0, o, jnp.exp(o) - 1.0)
        if layer1:
            la_ref[...] = la

    const = lambda i: (0, 0)
    row = lambda i: (i, 0)
    in_specs = [
        pl.BlockSpec((BR, SW), row),
        pl.BlockSpec((BR, SW), row),
        pl.BlockSpec((BR, HC), row),
        pl.BlockSpec((BR, HC), row),
        pl.BlockSpec((BR, HC), row),
    ]
    args = [a0, a1, S, xl, xr]
    if not layer1:
        in_specs.append(pl.BlockSpec((BR, 1), row))
        args.append(la_in)
    in_specs += [
        pl.BlockSpec((1, HC), const),
        pl.BlockSpec((1, HC), const),
        pl.BlockSpec((1, HC), const),
        pl.BlockSpec((HC, H), const),
        pl.BlockSpec((H, HC), const),
    ]
    args += [Wef, attf, biasf, sel2, seldn]
    out_specs = [pl.BlockSpec((BR, HC), row)]
    out_shape = [jax.ShapeDtypeStruct((N, HC), _F32)]
    if layer1:
        out_specs.append(pl.BlockSpec((BR, 1), row))
        out_shape.append(jax.ShapeDtypeStruct((N, 1), _F32))
    res = pl.pallas_call(body, grid=(N // BR,), in_specs=in_specs,
                         out_specs=out_specs, out_shape=out_shape)(*args)
    return res if layer1 else res[0]


def _head_mats(H, C):
    eye = jnp.eye(H, dtype=_F32)
    sel2 = jnp.repeat(eye, C, axis=0)
    seldn = jnp.repeat(eye, C, axis=1)
    return sel2, seldn


def kernel(x, edge_index, edge_attr, Wl1, bl1, Wr1, br1, We1, att1, bias1,
           Wl2, bl2, Wr2, br2, We2, att2, bias2):
    src = edge_index[0]
    dst = edge_index[1]
    eaf = edge_attr[:, 0]

    attf1 = att1.reshape(1, HC1)
    wef1 = We1.reshape(1, HC1)
    attf2 = att2.reshape(1, HC2)
    wef2 = We2.reshape(1, HC2)
    attsp1 = jnp.broadcast_to(att1.reshape(HC1, 1), (HC1, 16))
    wesp1 = jnp.broadcast_to(We1.reshape(HC1, 1), (HC1, 16))
    attsp2 = jnp.broadcast_to(att2.reshape(HC2, 1), (HC2, 16))
    wesp2 = jnp.broadcast_to(We2.reshape(HC2, 1), (HC2, 16))
    sel2_1, seldn_1 = _head_mats(H1, C1)
    sel2_2, seldn_2 = _head_mats(H2, C2)

    xl1, xr1 = _mm_pair(x, Wl1, bl1.reshape(1, HC1), Wr1, br1.reshape(1, HC1))
    accA1, w1 = _make_pass_a(H1, C1)(src, dst, eaf, xl1, xr1,
                                     attsp1, wesp1)
    s0 = xl1[:, 0:64]
    s1 = xl1[:, 64:96]
    s2 = xl1[:, 96:128]
    s3 = xl1[:, 128:192]
    accB1 = _make_pass_b(64, 32, 0, 2)(src, dst, s0, s3, s1, s2, w1)
    S1 = jnp.concatenate([accB1[0, :, 0:64], accB1[0, :, 64:96],
                          accB1[1, :, 64:96], accB1[1, :, 0:64]], axis=1)
    h1, la = _finalize(accA1[0], accA1[1], S1, xl1, xr1, None,
                       wef1, attf1, bias1.reshape(1, HC1),
                       sel2_1, seldn_1, H1, C1, True)

    xl2, xr2 = _mm_pair(h1, Wl2, bl2.reshape(1, HC2), Wr2, br2.reshape(1, HC2))
    accA2, w2 = _make_pass_a(H2, C2)(src, dst, eaf, xl2, xr2,
                                     attsp2, wesp2)
    t0 = xl2[:, 0:64]
    t1 = xl2[:, 64:128]
    accB2 = _make_pass_b(64, 0, 0, 0)(src, dst, t0, t1, t0, t1, w2)
    S2 = jnp.concatenate([accB2[0], accB2[1]], axis=1)
    h2 = _finalize(accA2[0], accA2[1], S2, xl2, xr2, la,
                   wef2, attf2, bias2.reshape(1, HC2),
                   sel2_2, seldn_2, H2, C2, False)
    return h2


# speedup vs baseline: 4.3075x; 1.0710x over previous
import functools

import jax
import jax.numpy as jnp
from jax import lax
from jax.experimental import pallas as pl
from jax.experimental.pallas import tpu as pltpu
from jax.experimental.pallas import tpu_sc as plsc

N = 10000
E = 320000
D = 128
H1, C1 = 3, 64
H2, C2 = 1, 128
HC1 = H1 * C1
HC2 = H2 * C2

NC, NS = 2, 16
NT = NC * NS
K = 80
J = K // 16
RPT8 = 624
REM = N - NS * RPT8
SW = 8

_F32 = jnp.float32
_SC_PARAMS = dict(use_tc_tiling_on_sc=False, needs_layout_passes=False)


def _zero_accum(stage, accum, sub):
    base = sub * RPT8
    for b in range(7):
        pltpu.sync_copy(stage, accum.at[pl.ds(base + b * 80, 80)])
    pltpu.sync_copy(stage.at[pl.ds(0, 64)], accum.at[pl.ds(base + 560, 64)])

    @pl.when(sub == 0)
    def _rem():
        pltpu.sync_copy(stage.at[pl.ds(0, REM)],
                        accum.at[pl.ds(NS * RPT8, REM)])


def _flush_accum(accum, out_hbm, stage, core, sub):
    base = sub * RPT8
    for b in range(7):
        pltpu.sync_copy(accum.at[pl.ds(base + b * 80, 80)], stage)
        pltpu.sync_copy(stage, out_hbm.at[core, pl.ds(base + b * 80, 80)])
    pltpu.sync_copy(accum.at[pl.ds(base + 560, 64)], stage.at[pl.ds(0, 64)])
    pltpu.sync_copy(stage.at[pl.ds(0, 64)],
                    out_hbm.at[core, pl.ds(base + 560, 64)])

    @pl.when(sub == 0)
    def _rem():
        pltpu.sync_copy(accum.at[pl.ds(NS * RPT8, REM)],
                        stage.at[pl.ds(0, REM)])
        pltpu.sync_copy(stage.at[pl.ds(0, REM)],
                        out_hbm.at[core, pl.ds(NS * RPT8, REM)])


def _pass_a_body(H, C,
                 src_hbm, dst_hbm, ea_hbm, xl_hbm, xr_hbm,
                 attsp_hbm, wesp_hbm, acc_hbm, w_hbm,
                 src_v, dst_v, ea_v, xl_rows, xr_rows, wsb,
                 attsp_v, wesp_v, accum, sem1, sem2):
    HC = H * C
    core = lax.axis_index("c")
    sub = lax.axis_index("s")
    wid = core * NS + sub
    ept = E // NT

    pltpu.sync_copy(attsp_hbm, attsp_v)
    pltpu.sync_copy(wesp_hbm, wesp_v)

    zero16 = jnp.zeros((16,), _F32)
    ones16 = jnp.ones((16,), _F32)
    iota16 = lax.iota(jnp.int32, 16)
    eids = [iota16 + (j * 16) for j in range(J)]

    for j in range(J):
        for col in range(SW):
            plsc.store_scatter(wsb, [eids[j], jnp.full((16,), col, jnp.int32)],
                               zero16)
    _zero_accum(wsb, accum, sub)

    plsc.subcore_barrier()

    ebase = wid * ept

    def chunk(g, carry):
        base = ebase + g * K
        ci1 = pltpu.async_copy(src_hbm.at[pl.ds(base, K)], src_v, sem1)
        ci2 = pltpu.async_copy(dst_hbm.at[pl.ds(base, K)], dst_v, sem2)
        ci3 = pltpu.async_copy(ea_hbm.at[pl.ds(base, K)], ea_v, sem1)
        ci1.wait()
        ci2.wait()
        ci3.wait()
        cp1 = pltpu.async_copy(xl_hbm.at[src_v], xl_rows, sem1)
        cp2 = pltpu.async_copy(xr_hbm.at[dst_v], xr_rows, sem2)
        cp1.wait()
        cp2.wait()

        eav = [ea_v[pl.ds(j * 16, 16)] for j in range(J)]

        for h in range(H):
            def cbody(ci, lg, _h=h):
                c = _h * C + ci
                cvec = jnp.full((16,), c, jnp.int32)
                attc = attsp_v[c]
                wec = wesp_v[c]
                out = []
                for j in range(J):
                    xlv = plsc.load_gather(xl_rows, [eids[j], cvec])
                    xrv = plsc.load_gather(xr_rows, [eids[j], cvec])
                    s = xlv + xrv + eav[j] * wec
                    lk = jnp.maximum(s, 0.2 * s)
                    out.append(lg[j] + attc * lk)
                return tuple(out)
            lg = lax.fori_loop(0, C, cbody, tuple(zero16 for _ in range(J)))
            hcol = jnp.full((16,), h, jnp.int32)
            for j in range(J):
                plsc.store_scatter(wsb, [eids[j], hcol], jnp.exp(lg[j]))

        eacol = jnp.full((16,), H, jnp.int32)
        onecol = jnp.full((16,), H + 1, jnp.int32)
        for j in range(J):
            plsc.store_scatter(wsb, [eids[j], eacol], eav[j])
            plsc.store_scatter(wsb, [eids[j], onecol], ones16)

        cw = pltpu.async_copy(wsb, w_hbm.at[pl.ds(base, K)], sem1)
        pltpu.sync_copy(wsb, accum.at[dst_v], add=True)
        cw.wait()
        return carry

    lax.fori_loop(0, ept // K, chunk, 0)

    plsc.subcore_barrier()
    _flush_accum(accum, acc_hbm, wsb, core, sub)


@functools.lru_cache(maxsize=None)
def _make_pass_a(H, C):
    HC = H * C
    mesh = plsc.VectorSubcoreMesh(core_axis_name="c", subcore_axis_name="s",
                                  num_cores=NC, num_subcores=NS)
    return pl.kernel(
        functools.partial(_pass_a_body, H, C),
        out_type=(jax.ShapeDtypeStruct((NC, N, SW), _F32),
                  jax.ShapeDtypeStruct((E, SW), _F32)),
        mesh=mesh,
        compiler_params=pltpu.CompilerParams(**_SC_PARAMS),
        scratch_types=[
            pltpu.VMEM((K,), jnp.int32),
            pltpu.VMEM((K,), jnp.int32),
            pltpu.VMEM((K,), _F32),
            pltpu.VMEM((K, HC), _F32),
            pltpu.VMEM((K, HC), _F32),
            pltpu.VMEM((K, SW), _F32),
            pltpu.VMEM((HC, 16), _F32),
            pltpu.VMEM((HC, 16), _F32),
            pltpu.VMEM_SHARED((N, SW), _F32),
            pltpu.SemaphoreType.DMA,
            pltpu.SemaphoreType.DMA,
        ],
    )


def _pass_b_body(WA, WB, wcol0, wcol1,
                 src_hbm, dst_hbm, ta0_hbm, ta1_hbm, tb0_hbm, tb1_hbm,
                 w_hbm, acc_hbm,
                 src_v, dst_v, bufa, bufb, wvb, contrib, accum, sem1, sem2):
    ACCW = WA + WB
    core = lax.axis_index("c")
    sub = lax.axis_index("s")
    ept = E // NS

    zero16 = jnp.zeros((16,), _F32)
    for r in range(K):
        for c0 in range(0, ACCW, 16):
            contrib[r, pl.ds(c0, 16)] = zero16
    _zero_accum(contrib, accum, sub)

    iota16 = lax.iota(jnp.int32, 16)
    eids = [iota16 + (j * 16) for j in range(J)]
    wca = jnp.where(core == 0, wcol0, wcol1)
    wcavec = jnp.full((16,), wca, jnp.int32)

    plsc.subcore_barrier()

    ebase = sub * ept

    def chunk(g, carry):
        base = ebase + g * K
        ci1 = pltpu.async_copy(src_hbm.at[pl.ds(base, K)], src_v, sem1)
        ci2 = pltpu.async_copy(dst_hbm.at[pl.ds(base, K)], dst_v, sem2)
        ci3 = pltpu.async_copy(w_hbm.at[pl.ds(base, K)], wvb, sem1)
        ci1.wait()
        ci2.wait()
        ci3.wait()

        @pl.when(core == 0)
        def _g0():
            cp1 = pltpu.async_copy(ta0_hbm.at[src_v], bufa, sem1)
            if WB:
                cp2 = pltpu.async_copy(tb0_hbm.at[src_v], bufb, sem2)
                cp2.wait()
            cp1.wait()

        @pl.when(core != 0)
        def _g1():
            cp1 = pltpu.async_copy(ta1_hbm.at[src_v], bufa, sem1)
            if WB:
                cp2 = pltpu.async_copy(tb1_hbm.at[src_v], bufb, sem2)
                cp2.wait()
            cp1.wait()

        wa = [plsc.load_gather(wvb, [eids[j], wcavec]) for j in range(J)]

        def abody(ci, carry2):
            cvec = jnp.full((16,), ci, jnp.int32)
            for j in range(J):
                xlv = plsc.load_gather(bufa, [eids[j], cvec])
                plsc.store_scatter(contrib, [eids[j], cvec], xlv * wa[j])
            return carry2
        lax.fori_loop(0, WA, abody, 0)

        if WB:
            wbv = jnp.full((16,), wcol1 if wcol0 == wcol1 else 1, jnp.int32)
            wb = [plsc.load_gather(wvb, [eids[j], wbv]) for j in range(J)]

            def bbody(ci, carry2):
                cvec = jnp.full((16,), ci, jnp.int32)
                ovec = jnp.full((16,), WA + ci, jnp.int32)
                for j in range(J):
                    xlv = plsc.load_gather(bufb, [eids[j], cvec])
                    plsc.store_scatter(contrib, [eids[j], ovec], xlv * wb[j])
                return carry2
            lax.fori_loop(0, WB, bbody, 0)

        pltpu.sync_copy(contrib, accum.at[dst_v], add=True)
        return carry

    lax.fori_loop(0, ept // K, chunk, 0)

    plsc.subcore_barrier()
    _flush_accum(accum, acc_hbm, contrib, core, sub)


@functools.lru_cache(maxsize=None)
def _make_pass_b(WA, WB, wcol0, wcol1):
    ACCW = WA + WB
    mesh = plsc.VectorSubcoreMesh(core_axis_name="c", subcore_axis_name="s",
                                  num_cores=NC, num_subcores=NS)
    return pl.kernel(
        functools.partial(_pass_b_body, WA, WB, wcol0, wcol1),
        out_type=jax.ShapeDtypeStruct((NC, N, ACCW), _F32),
        mesh=mesh,
        compiler_params=pltpu.CompilerParams(**_SC_PARAMS),
        scratch_types=[
            pltpu.VMEM((K,), jnp.int32),
            pltpu.VMEM((K,), jnp.int32),
            pltpu.VMEM((K, WA), _F32),
            pltpu.VMEM((K, max(WB, 8)), _F32),
            pltpu.VMEM((K, SW), _F32),
            pltpu.VMEM((K, ACCW), _F32),
            pltpu.VMEM_SHARED((N, ACCW), _F32),
            pltpu.SemaphoreType.DMA,
            pltpu.SemaphoreType.DMA,
        ],
    )


def _mm_pair(x, Wl, bl, Wr, br):
    M, Kd = x.shape
    HC = Wl.shape[1]
    BM = 1000

    def body(x_ref, wl_ref, bl_ref, wr_ref, br_ref, ol_ref, or_ref):
        xb = x_ref[...]
        ol_ref[...] = jnp.dot(xb, wl_ref[...],
                              preferred_element_type=_F32) + bl_ref[...]
        or_ref[...] = jnp.dot(xb, wr_ref[...],
                              preferred_element_type=_F32) + br_ref[...]

    return pl.pallas_call(
        body,
        grid=(M // BM,),
        in_specs=[
            pl.BlockSpec((BM, Kd), lambda i: (i, 0)),
            pl.BlockSpec((Kd, HC), lambda i: (0, 0)),
            pl.BlockSpec((1, HC), lambda i: (0, 0)),
            pl.BlockSpec((Kd, HC), lambda i: (0, 0)),
            pl.BlockSpec((1, HC), lambda i: (0, 0)),
        ],
        out_specs=[
            pl.BlockSpec((BM, HC), lambda i: (i, 0)),
            pl.BlockSpec((BM, HC), lambda i: (i, 0)),
        ],
        out_shape=[
            jax.ShapeDtypeStruct((M, HC), _F32),
            jax.ShapeDtypeStruct((M, HC), _F32),
        ],
    )(x, Wl, bl, Wr, br)


def _finalize(a0, a1, S, xl, xr, la_in, Wef, attf, biasf, sel2, seldn,
              H, C, layer1):
    HC = H * C
    BR = 1000

    def body(*refs):
        if layer1:
            (a0_ref, a1_ref, s_ref, xl_ref, xr_ref, wef_ref, attf_ref,
             b_ref, s2_ref, sd_ref, out_ref, la_ref) = refs
        else:
            (a0_ref, a1_ref, s_ref, xl_ref, xr_ref, lain_ref, wef_ref,
             attf_ref, b_ref, s2_ref, sd_ref, out_ref) = refs
        stats = a0_ref[...] + a1_ref[...]
        S = s_ref[...]
        Dh = stats[:, :H]
        if layer1:
            la = stats[:, H:H + 1] / jnp.maximum(stats[:, H + 1:H + 2], 1.0)
        else:
            la = lain_ref[...]
        xlb = xl_ref[...]
        m = xlb + xr_ref[...] + la * wef_ref[...]
        m = jnp.where(m > 0, m, 0.2 * m)
        a = jnp.dot(m * attf_ref[...], s2_ref[...],
                    preferred_element_type=_F32)
        wl = jnp.exp(a)
        dfull = jnp.dot(Dh + wl, sd_ref[...], preferred_element_type=_F32)
        wfull = jnp.dot(wl, sd_ref[...], preferred_element_type=_F32)
        o = (S + xlb * wfull) / (dfull + 1e-16) + b_ref[...]
        out_ref[...] = jnp.where(o > 0, o, jnp.exp(o) - 1.0)
        if layer1:
            la_ref[...] = la

    const = lambda i: (0, 0)
    row = lambda i: (i, 0)
    in_specs = [
        pl.BlockSpec((BR, SW), row),
        pl.BlockSpec((BR, SW), row),
        pl.BlockSpec((BR, HC), row),
        pl.BlockSpec((BR, HC), row),
        pl.BlockSpec((BR, HC), row),
    ]
    args = [a0, a1, S, xl, xr]
    if not layer1:
        in_specs.append(pl.BlockSpec((BR, 1), row))
        args.append(la_in)
    in_specs += [
        pl.BlockSpec((1, HC), const),
        pl.BlockSpec((1, HC), const),
        pl.BlockSpec((1, HC), const),
        pl.BlockSpec((HC, H), const),
        pl.BlockSpec((H, HC), const),
    ]
    args += [Wef, attf, biasf, sel2, seldn]
    out_specs = [pl.BlockSpec((BR, HC), row)]
    out_shape = [jax.ShapeDtypeStruct((N, HC), _F32)]
    if layer1:
        out_specs.append(pl.BlockSpec((BR, 1), row))
        out_shape.append(jax.ShapeDtypeStruct((N, 1), _F32))
    res = pl.pallas_call(body, grid=(N // BR,), in_specs=in_specs,
                         out_specs=out_specs, out_shape=out_shape)(*args)
    return res if layer1 else res[0]


def _head_mats(H, C):
    eye = jnp.eye(H, dtype=_F32)
    sel2 = jnp.repeat(eye, C, axis=0)
    seldn = jnp.repeat(eye, C, axis=1)
    return sel2, seldn


def kernel(x, edge_index, edge_attr, Wl1, bl1, Wr1, br1, We1, att1, bias1,
           Wl2, bl2, Wr2, br2, We2, att2, bias2):
    src = edge_index[0]
    dst = edge_index[1]
    eaf = edge_attr[:, 0]

    attf1 = att1.reshape(1, HC1)
    wef1 = We1.reshape(1, HC1)
    attf2 = att2.reshape(1, HC2)
    wef2 = We2.reshape(1, HC2)
    attsp1 = jnp.broadcast_to(att1.reshape(HC1, 1), (HC1, 16))
    wesp1 = jnp.broadcast_to(We1.reshape(HC1, 1), (HC1, 16))
    attsp2 = jnp.broadcast_to(att2.reshape(HC2, 1), (HC2, 16))
    wesp2 = jnp.broadcast_to(We2.reshape(HC2, 1), (HC2, 16))
    sel2_1, seldn_1 = _head_mats(H1, C1)
    sel2_2, seldn_2 = _head_mats(H2, C2)

    xl1, xr1 = _mm_pair(x, Wl1, bl1.reshape(1, HC1), Wr1, br1.reshape(1, HC1))
    accA1, w1 = _make_pass_a(H1, C1)(src, dst, eaf, xl1, xr1,
                                     attsp1, wesp1)
    s0 = xl1[:, 0:64]
    s1 = xl1[:, 64:96]
    s2 = xl1[:, 96:128]
    s3 = xl1[:, 128:192]
    accB1 = _make_pass_b(64, 32, 0, 2)(src, dst, s0, s3, s1, s2, w1)
    S1 = jnp.concatenate([accB1[0, :, 0:64], accB1[0, :, 64:96],
                          accB1[1, :, 64:96], accB1[1, :, 0:64]], axis=1)
    h1, la = _finalize(accA1[0], accA1[1], S1, xl1, xr1, None,
                       wef1, attf1, bias1.reshape(1, HC1),
                       sel2_1, seldn_1, H1, C1, True)

    xl2, xr2 = _mm_pair(h1, Wl2, bl2.reshape(1, HC2), Wr2, br2.reshape(1, HC2))
    accA2, w2 = _make_pass_a(H2, C2)(src, dst, eaf, xl2, xr2,
                                     attsp2, wesp2)
    t0 = xl2[:, 0:64]
    t1 = xl2[:, 64:128]
    accB2 = _make_pass_b(64, 0, 0, 0)(src, dst, t0, t1, t0, t1, w2)
    S2 = jnp.concatenate([accB2[0], accB2[1]], axis=1)
    h2 = _finalize(accA2[0], accA2[1], S2, xl2, xr2, la,
                   wef2, attf2, bias2.reshape(1, HC2),
                   sel2_2, seldn_2, H2, C2, False)
    return h2
